# Initial kernel scaffold; baseline (speedup 1.0000x reference)
import jax
import jax.numpy as jnp
from jax.experimental import pallas as pl


def kernel(x, edge_index, edge_attr, batch, params):
    raise NotImplementedError("write your pallas kernel here")


# speedup vs baseline: 2.2984x; 2.2984x over previous
import functools

import jax
import jax.numpy as jnp
from jax import lax
from jax.experimental import pallas as pl
from jax.experimental.pallas import tpu as pltpu
from jax.experimental.pallas import tpu_sc as plsc

N = 10000
NP = 10240
G = 64
DE = 16
NC, NS, LN = 2, 16, 16
NW = NC * NS
B = 128
NBLK = 80
EPT = NBLK * B
EP = NW * EPT
RPT = NP // NS
RB = 256
NBT = NP // RB
EPS = 1e-5
DIMS = [(128, 512), (512, 256), (256, 128), (128, 256), (256, 512)]

@functools.lru_cache(maxsize=None)
def _mesh():
    return plsc.VectorSubcoreMesh(core_axis_name="c", subcore_axis_name="s",
                                  num_cores=NC, num_subcores=NS)


def _sc_deg_body(dst_hbm, ones_hbm, zero_hbm, out_hbm, dst_v, ones_v, zv, acc):
    c = lax.axis_index("c")
    s = lax.axis_index("s")
    wid = s * NC + c
    pltpu.sync_copy(dst_hbm.at[wid], dst_v)
    pltpu.sync_copy(ones_hbm, ones_v)
    pltpu.sync_copy(zero_hbm, zv)
    for t in range(RPT // B):
        pltpu.sync_copy(zv, acc.at[pl.ds(s * RPT + t * B, B)])
    plsc.subcore_barrier()

    for j in range(NBLK):
        pltpu.sync_copy(ones_v, acc.at[dst_v.at[j]], add=True)
    plsc.subcore_barrier()
    for t in range(RPT // B):
        pltpu.sync_copy(acc.at[pl.ds(s * RPT + t * B, B)], zv)
        pltpu.sync_copy(zv, out_hbm.at[c, pl.ds(s * RPT + t * B, B)])


@functools.lru_cache(maxsize=None)
def _sc_deg_kernel():
    return pl.kernel(
        _sc_deg_body,
        out_type=jax.ShapeDtypeStruct((NC, NP, B), jnp.float32),
        mesh=_mesh(),
        compiler_params=pltpu.CompilerParams(needs_layout_passes=False),
        scratch_types=[
            pltpu.VMEM((NBLK, B), jnp.int32),
            pltpu.VMEM((B, B), jnp.float32),
            pltpu.VMEM((B, B), jnp.float32),
            pltpu.VMEM_SHARED((NP, B), jnp.float32),
        ],
    )


def _sc_deg(*args):
    return _sc_deg_kernel()(*args)


def _sc_qgather_body(dinvb_hbm, src_hbm, out_hbm, src_v, r0, r1, sem0, sem1):
    c = lax.axis_index("c")
    s = lax.axis_index("s")
    wid = s * NC + c
    pltpu.sync_copy(src_hbm.at[wid], src_v)
    rows = (r0, r1)
    sems = (sem0, sem1)
    cps = [pltpu.async_copy(dinvb_hbm.at[src_v.at[0]], r0, sem0), None]
    for j in range(NBLK):
        slot = j % 2
        cps[slot].wait()
        if j + 1 < NBLK:
            nslot = (j + 1) % 2
            cps[nslot] = pltpu.async_copy(
                dinvb_hbm.at[src_v.at[j + 1]], rows[nslot], sems[nslot])
        pltpu.sync_copy(rows[slot], out_hbm.at[wid, pl.ds(j * B, B)])


@functools.lru_cache(maxsize=None)
def _sc_qgather_kernel():
    return pl.kernel(
        _sc_qgather_body,
        out_type=jax.ShapeDtypeStruct((NW, EPT, B), jnp.float32),
        mesh=_mesh(),
        compiler_params=pltpu.CompilerParams(needs_layout_passes=False),
        scratch_types=[
            pltpu.VMEM((NBLK, B), jnp.int32),
            pltpu.VMEM((B, B), jnp.float32),
            pltpu.VMEM((B, B), jnp.float32),
            pltpu.SemaphoreType.DMA,
            pltpu.SemaphoreType.DMA,
        ],
    )


def _sc_scatter16_body(sea_hbm, dst_hbm, zero_hbm, out_hbm,
                       dst_v, ea_v, zv, acc):
    c = lax.axis_index("c")
    s = lax.axis_index("s")
    wid = s * NC + c
    pltpu.sync_copy(dst_hbm.at[wid], dst_v)
    pltpu.sync_copy(zero_hbm, zv)
    for t in range(RPT // B):
        pltpu.sync_copy(zv, acc.at[pl.ds(s * RPT + t * B, B)])
    plsc.subcore_barrier()
    for j in range(NBLK):
        pltpu.sync_copy(sea_hbm.at[wid, pl.ds(j * B, B)], ea_v)
        pltpu.sync_copy(ea_v, acc.at[dst_v.at[j]], add=True)
    plsc.subcore_barrier()
    for t in range(RPT // B):
        pltpu.sync_copy(acc.at[pl.ds(s * RPT + t * B, B)], zv)
        pltpu.sync_copy(zv, out_hbm.at[c, pl.ds(s * RPT + t * B, B)])


@functools.lru_cache(maxsize=None)
def _sc_scatter16_kernel():
    return pl.kernel(
        _sc_scatter16_body,
        out_type=jax.ShapeDtypeStruct((NC, NP, B), jnp.float32),
        mesh=_mesh(),
        compiler_params=pltpu.CompilerParams(needs_layout_passes=False),
        scratch_types=[
            pltpu.VMEM((NBLK, B), jnp.int32),
            pltpu.VMEM((B, B), jnp.float32),
            pltpu.VMEM((B, B), jnp.float32),
            pltpu.VMEM_SHARED((NP, B), jnp.float32),
        ],
    )


def _tc_scale_body(q_ref, e_ref, o_ref):
    o_ref[...] = jnp.concatenate(
        [q_ref[:, 0:1] * e_ref[...],
         jnp.zeros((_SCALE_RB, B - DE), jnp.float32)], axis=1)


_SCALE_RB = 2048


def _tc_scale(qf, ea2):
    return pl.pallas_call(
        _tc_scale_body,
        grid=(EP // _SCALE_RB,),
        in_specs=[_rows((_SCALE_RB, B)), _rows((_SCALE_RB, DE))],
        out_specs=_rows((_SCALE_RB, B)),
        out_shape=jax.ShapeDtypeStruct((EP, B), jnp.float32),
    )(qf, ea2)


def _sc_araw(src3, dst3, eap, dinvb, zero_bb):
    qf = _sc_qgather_kernel()(dinvb, src3)
    sea = _tc_scale(qf.reshape(EP, B), eap)
    return _sc_scatter16_kernel()(sea.reshape(NW, EPT, B), dst3, zero_bb)


NGI = 16


def _sc_spmm_body(h_hbm, src_hbm, dst_hbm, zero_hbm, out_hbm,
                  src_v, dst_v, r0, r1, sem0, sem1, acc):
    c = lax.axis_index("c")
    s = lax.axis_index("s")
    wid = s * NC + c
    pltpu.sync_copy(zero_hbm, r0)
    for t in range(RPT // B):
        pltpu.sync_copy(r0, acc.at[pl.ds(s * RPT + t * B, B)])
    plsc.subcore_barrier()

    rows = (r0, r1)
    sems = (sem0, sem1)
    for g in range(NBLK // NGI):
        pltpu.sync_copy(src_hbm.at[wid, pl.ds(g * NGI, NGI)], src_v)
        pltpu.sync_copy(dst_hbm.at[wid, pl.ds(g * NGI, NGI)], dst_v)
        cps = [pltpu.async_copy(h_hbm.at[src_v.at[0]], r0, sem0), None]
        for j in range(NGI):
            slot = j % 2
            cps[slot].wait()
            if j + 1 < NGI:
                nslot = (j + 1) % 2
                cps[nslot] = pltpu.async_copy(
                    h_hbm.at[src_v.at[j + 1]], rows[nslot], sems[nslot])
            pltpu.sync_copy(rows[slot], acc.at[dst_v.at[j]], add=True)
    plsc.subcore_barrier()
    for t in range(RPT // B):
        pltpu.sync_copy(acc.at[pl.ds(s * RPT + t * B, B)], r0)
        pltpu.sync_copy(r0, out_hbm.at[c, pl.ds(s * RPT + t * B, B)])


@functools.lru_cache(maxsize=None)
def _sc_spmm_kernel():
    return pl.kernel(
        _sc_spmm_body,
        out_type=jax.ShapeDtypeStruct((NC, NP, B), jnp.float32),
        mesh=_mesh(),
        compiler_params=pltpu.CompilerParams(needs_layout_passes=False),
        scratch_types=[
            pltpu.VMEM((NGI, B), jnp.int32),
            pltpu.VMEM((NGI, B), jnp.int32),
            pltpu.VMEM((B, B), jnp.float32),
            pltpu.VMEM((B, B), jnp.float32),
            pltpu.SemaphoreType.DMA,
            pltpu.SemaphoreType.DMA,
            pltpu.VMEM_SHARED((NP, B), jnp.float32),
        ],
    )


def _sc_spmm(*args):
    return _sc_spmm_kernel()(*args)


def _rows_valid(i):
    r = lax.broadcasted_iota(jnp.int32, (RB, 1), 0) + i * RB
    return r < N


def _tc_init_body(deg_ref, x_ref, w_ref, dinv_ref, *h_refs):
    i = pl.program_id(0)
    d = deg_ref[0, :, 0:1] + deg_ref[1, :, 0:1]
    dinv = jnp.where(_rows_valid(i), lax.rsqrt(jnp.clip(d, 1.0, None)), 0.0)
    dinv_ref[...] = jnp.broadcast_to(dinv, (RB, B))
    h = dinv * jnp.dot(x_ref[...], w_ref[...],
                       preferred_element_type=jnp.float32)
    for ci, hr in enumerate(h_refs):
        hr[...] = h[:, ci * B:(ci + 1) * B]


def _tc_gcn_a_body(nc, araw_ref, dinv_ref, we_ref, b_ref, *rest):
    chunk_refs = rest[:nc]
    y_ref, st_ref = rest[nc], rest[nc + 1]
    i = pl.program_id(0)
    dinv = dinv_ref[:, 0:1]
    ar = araw_ref[0, :, :DE] + araw_ref[1, :, :DE]
    et = jnp.dot(ar, we_ref[...], preferred_element_type=jnp.float32)
    sc = jnp.concatenate([r[0] + r[1] for r in chunk_refs], axis=1)
    z = dinv * (sc + et) + b_ref[...]
    y = jnp.where(z >= 0, z, 0.01 * z)
    y_ref[...] = y
    ym = jnp.where(_rows_valid(i), y, 0.0)
    st = jnp.concatenate([jnp.sum(ym, axis=0, keepdims=True),
                          jnp.sum(ym * ym, axis=0, keepdims=True)], axis=0)
    st_ref[...] = jnp.where(i == 0, st, st_ref[...] + st)


def _bn(y, st_ref, g_ref, beta_ref):
    mu = st_ref[0:1] / N
    var = st_ref[1:2] / N - mu * mu
    return (y - mu) * lax.rsqrt(var + EPS) * g_ref[...] + beta_ref[...]


def _tc_gcn_b_body(y_ref, st_ref, g_ref, beta_ref, w_ref, dinv_ref, *h_refs):
    x = _bn(y_ref[...], st_ref, g_ref, beta_ref)
    h = dinv_ref[:, 0:1] * jnp.dot(x, w_ref[...],
                                   preferred_element_type=jnp.float32)
    for ci, hr in enumerate(h_refs):
        hr[...] = h[:, ci * B:(ci + 1) * B]


def _tc_head1_body(y_ref, st_ref, g_ref, beta_ref, wg_ref, bg_ref, batch_ref,
                   x4_ref, lg_ref, m_ref):
    i = pl.program_id(0)
    x = _bn(y_ref[...], st_ref, g_ref, beta_ref)
    x4_ref[...] = x
    lg = jnp.dot(x, wg_ref[...], preferred_element_type=jnp.float32) \
        + bg_ref[...]
    lg_ref[...] = lg
    oh = batch_ref[...] == lax.broadcasted_iota(jnp.int32, (1, G), 1)
    mm = jnp.max(jnp.where(oh, jnp.broadcast_to(lg, (RB, G)), -jnp.inf),
                 axis=0, keepdims=True)
    m_ref[...] = jnp.where(i == 0, mm, jnp.maximum(m_ref[...], mm))


def _tc_head2_body(x4_ref, lg_ref, m_ref, batch_ref,
                   w1_ref, b1_ref, w2_ref, b2_ref, w3_ref, b3_ref,
                   o_ref, p_acc, s_acc):
    i = pl.program_id(0)
    m = m_ref[...]
    mfix = jnp.where(jnp.abs(m) < jnp.inf, m, 0.0)
    oh = (batch_ref[...] ==
          lax.broadcasted_iota(jnp.int32, (1, G), 1)).astype(jnp.float32)
    mi = jax.lax.dot_general(oh, mfix, (((1,), (1,)), ((), ())),
                             preferred_element_type=jnp.float32)
    e = jnp.where(_rows_valid(i), jnp.exp(lg_ref[...] - mi), 0.0)
    ps = jax.lax.dot_general(oh, e * x4_ref[...], (((0,), (0,)), ((), ())),
                             preferred_element_type=jnp.float32)
    ss = jnp.sum(oh * e, axis=0, keepdims=True)
    first = i == 0
    p_acc[...] = jnp.where(first, ps, p_acc[...] + ps)
    s_acc[...] = jnp.where(first, ss, s_acc[...] + ss)

    @pl.when(i == NBT - 1)
    def _():
        eye = jnp.eye(G, dtype=jnp.float32)
        rec = 1.0 / jnp.clip(s_acc[...], 1e-12, None)
        rec_t = jax.lax.dot_general(eye, rec, (((1,), (1,)), ((), ())),
                                    preferred_element_type=jnp.float32)
        pooled = p_acc[...] * rec_t
        o = jnp.dot(pooled, w1_ref[...],
                    preferred_element_type=jnp.float32) + b1_ref[...]
        o = jnp.where(o >= 0, o, 0.01 * o)
        o = jnp.dot(o, w2_ref[...],
                    preferred_element_type=jnp.float32) + b2_ref[...]
        o = jnp.where(o >= 0, o, 0.01 * o)
        o = jnp.dot(o, w3_ref[...],
                    preferred_element_type=jnp.float32) + b3_ref[...]
        o_ref[...] = o


def _full(shape):
    nd = len(shape)
    return pl.BlockSpec(shape, lambda i, _nd=nd: (0,) * _nd)


def _rows(shape):
    nd = len(shape)
    return pl.BlockSpec(shape, lambda i, _nd=nd: (0,) * (_nd - 2) + (i, 0))


def _tc_init(deg2, xp, w0):
    nc = DIMS[0][1] // B
    return pl.pallas_call(
        _tc_init_body,
        grid=(NBT,),
        in_specs=[pl.BlockSpec((NC, RB, B), lambda i: (0, i, 0)),
                  _rows((RB, 128)), _full((128, DIMS[0][1]))],
        out_specs=[_rows((RB, B))] + [_rows((RB, B))] * nc,
        out_shape=[jax.ShapeDtypeStruct((NP, B), jnp.float32)]
        + [jax.ShapeDtypeStruct((NP, B), jnp.float32)] * nc,
    )(deg2, xp, w0)


def _tc_gcn_a(chunks, araw2, dinvb, we, b):
    nc = len(chunks)
    do = nc * B
    body = functools.partial(_tc_gcn_a_body, nc)
    return pl.pallas_call(
        body,
        grid=(NBT,),
        in_specs=[pl.BlockSpec((NC, RB, B), lambda i: (0, i, 0)),
                  _rows((RB, B)), _full((DE, do)), _full((1, do))]
        + [pl.BlockSpec((NC, RB, B), lambda i: (0, i, 0))] * nc,
        out_specs=[_rows((RB, do)), _full((NC, do))],
        out_shape=[jax.ShapeDtypeStruct((NP, do), jnp.float32),
                   jax.ShapeDtypeStruct((NC, do), jnp.float32)],
    )(araw2, dinvb, we, b, *chunks)


def _tc_gcn_b(y, st, g, beta, w, dinvb):
    do = y.shape[1]
    dn = w.shape[1]
    nc = dn // B
    return pl.pallas_call(
        _tc_gcn_b_body,
        grid=(NBT,),
        in_specs=[_rows((RB, do)), _full((NC, do)), _full((1, do)),
                  _full((1, do)), _full((do, dn)), _rows((RB, B))],
        out_specs=[_rows((RB, B))] * nc,
        out_shape=[jax.ShapeDtypeStruct((NP, B), jnp.float32)] * nc,
    )(y, st, g, beta, w, dinvb)


def _tc_head1(y, st, g, beta, wg, bg, batch2):
    do = y.shape[1]
    return pl.pallas_call(
        _tc_head1_body,
        grid=(NBT,),
        in_specs=[_rows((RB, do)), _full((NC, do)), _full((1, do)),
                  _full((1, do)), _full((do, 1)), _full((1, 1)),
                  _rows((RB, 1))],
        out_specs=[_rows((RB, do)), _rows((RB, 1)), _full((1, G))],
        out_shape=[jax.ShapeDtypeStruct((NP, do), jnp.float32),
                   jax.ShapeDtypeStruct((NP, 1), jnp.float32),
                   jax.ShapeDtypeStruct((1, G), jnp.float32)],
    )(y, st, g, beta, wg, bg, batch2)


def _tc_head2(x4, lg, m, batch2, w1, b1, w2, b2, w3, b3):
    do = x4.shape[1]
    return pl.pallas_call(
        _tc_head2_body,
        grid=(NBT,),
        in_specs=[_rows((RB, do)), _rows((RB, 1)), _full((1, G)),
                  _rows((RB, 1)), _full((512, 256)), _full((1, 256)),
                  _full((256, 128)), _full((1, 128)), _full((128, 1)),
                  _full((1, 1))],
        out_specs=_full((G, 1)),
        out_shape=jax.ShapeDtypeStruct((G, 1), jnp.float32),
        scratch_shapes=[pltpu.VMEM((G, do), jnp.float32),
                        pltpu.VMEM((1, G), jnp.float32)],
    )(x4, lg, m, batch2, w1, b1, w2, b2, w3, b3)


def kernel(x, edge_index, edge_attr, batch, params):
    p = params
    f32 = jnp.float32
    src = jnp.concatenate(
        [edge_index[0], jnp.full((EP - edge_index.shape[1],), N, jnp.int32)])
    dst = jnp.concatenate(
        [edge_index[1], jnp.full((EP - edge_index.shape[1],), N, jnp.int32)])
    eap = jnp.concatenate(
        [edge_attr, jnp.zeros((EP - edge_attr.shape[0], DE), f32)])
    src3 = src.reshape(NW, NBLK, B)
    dst3 = dst.reshape(NW, NBLK, B)
    xp = jnp.concatenate([x, jnp.zeros((NP - N, x.shape[1]), f32)])
    batch2 = jnp.concatenate(
        [batch, jnp.full((NP - N,), G, jnp.int32)]).reshape(NP, 1)
    ones_b = jnp.ones((B, B), f32)
    zero_bb = jnp.zeros((B, B), f32)

    deg2 = _sc_deg(dst3, ones_b, zero_bb)
    init_out = _tc_init(deg2, xp, p["W0"])
    dinvb, chunks = init_out[0], list(init_out[1:])

    araw2 = _sc_araw(src3, dst3, eap, dinvb, zero_bb)

    y = None
    for l in range(5):
        do = DIMS[l][1]
        s_chunks = [_sc_spmm(hc, src3, dst3, zero_bb) for hc in chunks]
        y, st = _tc_gcn_a(s_chunks, araw2,
                          dinvb, p["We%d" % l].astype(f32),
                          p["b%d" % l].reshape(1, do))
        if l < 4:
            chunks = list(_tc_gcn_b(y, st, p["g%d" % l].reshape(1, do),
                                    p["beta%d" % l].reshape(1, do),
                                    p["W%d" % (l + 1)], dinvb))
        else:
            x4, lg, m = _tc_head1(y, st, p["g4"].reshape(1, do),
                                  p["beta4"].reshape(1, do),
                                  p["Wg"], p["bg"].reshape(1, 1), batch2)
            o = _tc_head2(x4, lg, m, batch2,
                          p["Wf1"], p["bf1"].reshape(1, 256),
                          p["Wf2"], p["bf2"].reshape(1, 128),
                          p["Wf3"], p["bf3"].reshape(1, 1))
    return o.reshape(-1)
